# Initial kernel scaffold; baseline (speedup 1.0000x reference)
import jax
import jax.numpy as jnp
from jax.experimental import pallas as pl


def kernel(text, emb_weight, fc_weight, fc_bias):
    raise NotImplementedError("write your pallas kernel here")


# speedup vs baseline: 2.5238x; 2.5238x over previous
import functools

import jax
import jax.numpy as jnp
from jax import lax
from jax.experimental import pallas as pl
from jax.experimental.pallas import tpu as pltpu
from jax.experimental.pallas import tpu_sc as plsc

VOCAB = 1000000
D = 64
NUM_CLASS = 20
B = 16384
HIST = 50

NC = 2
NS = 16
LANES = 16
NW = NC * NS
BPW = B // NW
BAGS_PER_CHUNK = 2
CIDX = BAGS_PER_CHUNK * HIST
NCH = BPW // BAGS_PER_CHUNK
ROW_BYTES = CIDX * D * 4


def _sc_body(text_hbm, table_hbm, dummy_hbm, out_hbm, idx_v, rows0, rows1, out_v,
             sem0, sem1):
    wid = lax.axis_index("s") * NC + lax.axis_index("c")

    pltpu.sync_copy(text_hbm.at[wid], idx_v)

    def start(chunk, rows, sem):
        pltpu.async_copy(table_hbm.at[idx_v.at[chunk]], rows, sem)

    def drain(rows, sem):
        pltpu.make_async_copy(dummy_hbm, rows, sem).wait()

    def accumulate(rows, chunk):
        for bag in range(BAGS_PER_CHUNK):
            def row_body(j, accs):
                r = bag * HIST + j
                return tuple(
                    accs[k] + rows[r, pl.ds(k * LANES, LANES)]
                    for k in range(D // LANES)
                )
            accs = tuple(jnp.zeros((LANES,), jnp.float32) for _ in range(D // LANES))
            accs = lax.fori_loop(0, HIST, row_body, accs)
            b_local = chunk * BAGS_PER_CHUNK + bag
            for k in range(D // LANES):
                out_v[b_local, pl.ds(k * LANES, LANES)] = accs[k]

    start(0, rows0, sem0)
    start(1, rows1, sem1)

    def body(c, _):
        drain(rows0, sem0)
        accumulate(rows0, c)

        @pl.when(c + 2 < NCH)
        def _():
            start(c + 2, rows0, sem0)

        drain(rows1, sem1)
        accumulate(rows1, c + 1)

        @pl.when(c + 3 < NCH)
        def _():
            start(c + 3, rows1, sem1)
        return 0

    lax.fori_loop(0, NCH // 2, lambda i, carry: body(i * 2, carry), 0)

    pltpu.sync_copy(out_v, out_hbm.at[wid])


_sc_sum = functools.partial(
    pl.kernel,
    mesh=plsc.VectorSubcoreMesh(core_axis_name="c", subcore_axis_name="s",
                                num_cores=NC, num_subcores=NS),
    out_type=jax.ShapeDtypeStruct((NW, BPW, D), jnp.float32),
    scratch_types=[
        pltpu.VMEM((NCH, CIDX), jnp.int32),
        pltpu.VMEM((CIDX, D), jnp.float32),
        pltpu.VMEM((CIDX, D), jnp.float32),
        pltpu.VMEM((BPW, D), jnp.float32),
        pltpu.SemaphoreType.DMA,
        pltpu.SemaphoreType.DMA,
    ],
    compiler_params=pltpu.CompilerParams(use_tc_tiling_on_sc=False),
)(_sc_body)


def _lin_body(x_ref, w_ref, b_ref, o_ref):
    o_ref[...] = (
        jnp.dot(x_ref[...], w_ref[...], preferred_element_type=jnp.float32)
        * (1.0 / HIST)
        + b_ref[...]
    )


_BLK = 2048
_linear = pl.pallas_call(
    _lin_body,
    grid=(B // _BLK,),
    in_specs=[
        pl.BlockSpec((_BLK, D), lambda i: (i, 0)),
        pl.BlockSpec((D, NUM_CLASS), lambda i: (0, 0)),
        pl.BlockSpec((1, NUM_CLASS), lambda i: (0, 0)),
    ],
    out_specs=pl.BlockSpec((_BLK, NUM_CLASS), lambda i: (i, 0)),
    out_shape=jax.ShapeDtypeStruct((B, NUM_CLASS), jnp.float32),
)


def kernel(text, emb_weight, fc_weight, fc_bias):
    idx = text.astype(jnp.int32).reshape(NW, NCH, CIDX)
    dummy = jnp.zeros((CIDX, D), jnp.float32)
    sums = _sc_sum(idx, emb_weight, dummy).reshape(B, D)
    return _linear(sums, fc_weight.T, fc_bias.reshape(1, NUM_CLASS))


# speedup vs baseline: 3.7018x; 1.4667x over previous
import functools

import jax
import jax.numpy as jnp
from jax import lax
from jax.experimental import pallas as pl
from jax.experimental.pallas import tpu as pltpu
from jax.experimental.pallas import tpu_sc as plsc

VOCAB = 1000000
D = 64
NUM_CLASS = 20
B = 16384
HIST = 50

K = 128
KACC = 32

NC = 2
NS = 16
LANES = 16
NW = NC * NS
BPW = B // NW
BAGS_PER_CHUNK = 2
CIDX = BAGS_PER_CHUNK * HIST
NCH = BPW // BAGS_PER_CHUNK


def _proj_body(et_ref, w_ref, b_ref, o_ref):
    o_ref[...] = (
        lax.dot_general(et_ref[...], w_ref[...], (((0,), (0,)), ((), ())),
                        preferred_element_type=jnp.float32)
        + b_ref[...]
    )


_VB = 8192
_proj = pl.pallas_call(
    _proj_body,
    grid=(pl.cdiv(VOCAB, _VB),),
    in_specs=[
        pl.BlockSpec((D, _VB), lambda i: (0, i)),
        pl.BlockSpec((D, K), lambda i: (0, 0)),
        pl.BlockSpec((1, K), lambda i: (0, 0)),
    ],
    out_specs=pl.BlockSpec((_VB, K), lambda i: (i, 0)),
    out_shape=jax.ShapeDtypeStruct((VOCAB, K), jnp.float32),
)


def _sc_body(text_hbm, table_hbm, dummy_hbm, out_hbm, idx_v, rows0, rows1, out_v,
             sem0, sem1):
    wid = lax.axis_index("s") * NC + lax.axis_index("c")

    pltpu.sync_copy(text_hbm.at[wid], idx_v)

    def start(chunk, rows, sem):
        pltpu.async_copy(table_hbm.at[idx_v.at[chunk]], rows, sem)

    def drain(rows, sem):
        pltpu.make_async_copy(dummy_hbm, rows, sem).wait()

    def accumulate(rows, chunk):
        for bag in range(BAGS_PER_CHUNK):
            def row_body(j, accs):
                r = bag * HIST + j
                return tuple(
                    accs[k] + rows[r, pl.ds(k * LANES, LANES)]
                    for k in range(KACC // LANES)
                )
            accs = tuple(jnp.zeros((LANES,), jnp.float32)
                         for _ in range(KACC // LANES))
            accs = lax.fori_loop(0, HIST, row_body, accs)
            b_local = chunk * BAGS_PER_CHUNK + bag
            for k in range(KACC // LANES):
                out_v[b_local, pl.ds(k * LANES, LANES)] = accs[k]

    start(0, rows0, sem0)
    start(1, rows1, sem1)

    def body(c, _):
        drain(rows0, sem0)
        accumulate(rows0, c)

        @pl.when(c + 2 < NCH)
        def _():
            start(c + 2, rows0, sem0)

        drain(rows1, sem1)
        accumulate(rows1, c + 1)

        @pl.when(c + 3 < NCH)
        def _():
            start(c + 3, rows1, sem1)
        return 0

    lax.fori_loop(0, NCH // 2, lambda i, carry: body(i * 2, carry), 0)

    pltpu.sync_copy(out_v, out_hbm.at[wid])


_sc_sum = functools.partial(
    pl.kernel,
    mesh=plsc.VectorSubcoreMesh(core_axis_name="c", subcore_axis_name="s",
                                num_cores=NC, num_subcores=NS),
    out_type=jax.ShapeDtypeStruct((NW, BPW, KACC), jnp.float32),
    scratch_types=[
        pltpu.VMEM((NCH, CIDX), jnp.int32),
        pltpu.VMEM((CIDX, K), jnp.float32),
        pltpu.VMEM((CIDX, K), jnp.float32),
        pltpu.VMEM((BPW, KACC), jnp.float32),
        pltpu.SemaphoreType.DMA,
        pltpu.SemaphoreType.DMA,
    ],
    compiler_params=pltpu.CompilerParams(use_tc_tiling_on_sc=False),
)(_sc_body)


def kernel(text, emb_weight, fc_weight, fc_bias):
    w128 = jnp.zeros((D, K), jnp.float32).at[:, :NUM_CLASS].set(
        fc_weight.T * (1.0 / HIST))
    b128 = jnp.zeros((1, K), jnp.float32).at[:, :NUM_CLASS].set(
        fc_bias.reshape(1, NUM_CLASS) * (1.0 / HIST))
    table = _proj(emb_weight.T, w128, b128)
    idx = text.astype(jnp.int32).reshape(NW, NCH, CIDX)
    dummy = jnp.zeros((CIDX, K), jnp.float32)
    sums = _sc_sum(idx, table, dummy).reshape(B, KACC)
    return sums[:, :NUM_CLASS]
